# baseline (speedup 1.0000x reference)
import jax
import jax.numpy as jnp
from jax import lax
from jax.experimental import pallas as pl
from jax.experimental.pallas import tpu as pltpu
from jax.experimental.pallas import tpu_sc as plsc

_NC = 2
_NS = 16
_NW = _NC * _NS
_C = 80
_G = 16
_NBUF = 5
_NSC = 2
_NIB = 8
_GPD = 3
_IPD = 6


def _ab_kernel(h_ref, w_ref, b_ref, d_ref, out_ref, hp_ref):
    ab = lax.dot_general(
        w_ref[...], h_ref[...], (((1,), (1,)), ((), ())),
        preferred_element_type=jnp.float32)
    out_ref[0:2, :] = ab + b_ref[...]
    out_ref[2, :] = d_ref[...]
    hb = h_ref[...].astype(jnp.bfloat16)
    d2 = hb.shape[1] // 2
    lo = lax.bitcast_convert_type(hb[:, :d2], jnp.uint16).astype(jnp.uint32)
    hi = lax.bitcast_convert_type(hb[:, d2:], jnp.uint16).astype(jnp.uint32)
    hp_ref[...] = lax.bitcast_convert_type((hi << 16) | lo, jnp.int32)


def _combine_kernel(p_ref, out_ref):
    out_ref[...] = p_ref[0] + p_ref[1]


def _make_gate_kernel(n, ept, nedges):
    nchunk = ept // _C

    def _gate_kernel(abd_hbm, ei_hbm, out3_hbm, abd_v, idx_v, out_v, sem):
        cid = lax.axis_index("c")
        sid = lax.axis_index("s")
        wid = sid * _NC + cid

        off = pl.multiple_of(wid * ept, 8)
        cp1 = pltpu.make_async_copy(abd_hbm, abd_v, sem)
        cp2 = pltpu.make_async_copy(
            ei_hbm.at[pl.ds(off, ept)], idx_v.at[pl.ds(0, ept)], sem)
        cp3 = pltpu.make_async_copy(
            ei_hbm.at[pl.ds(nedges + off, ept)], idx_v.at[pl.ds(ept, ept)],
            sem)
        cp1.start()
        cp2.start()
        cp3.start()
        cp1.wait()
        cp2.wait()
        cp3.wait()

        def chunk(c, carry):
            for g in range(_C // _G):
                sl = pl.ds(g * _G, _G)
                si = idx_v[pl.ds(c * _C + g * _G, _G)]
                di = idx_v[pl.ds(ept + c * _C + g * _G, _G)]
                av = plsc.load_gather(abd_v, [di])
                bv = plsc.load_gather(abd_v, [si + n])
                dd = plsc.load_gather(abd_v, [di + 2 * n])
                ds_ = plsc.load_gather(abd_v, [si + 2 * n])
                u = jnp.exp(2.0 * (av + bv))
                t = 1.0 - 2.0 / (u + 1.0)
                ev = t * dd * ds_
                out_v[c, 0, sl] = si
                out_v[c, 1, sl] = di
                out_v[c, 2, sl] = plsc.bitcast(ev, jnp.int32)
            return carry

        lax.fori_loop(0, nchunk, chunk, 0, unroll=2)
        pltpu.sync_copy(out_v, out3_hbm.at[wid])

    return _gate_kernel


def _make_scatter_kernel(n, d_feat, ept):
    nchunk = ept // _C
    rpt = (n // _NS) // 8 * 8
    rem = n - rpt * _NS

    zrows = n // _NS
    nzc = zrows // _C
    zrem = zrows - nzc * _C

    def _scatter_kernel(hp_hbm, eidx_hbm, out_hbm,
                        idx_v, rows_v, rowsf_v, z_sh, sem_i, sem_g, sem_s):
        cid = lax.axis_index("c")
        sid = lax.axis_index("s")
        wid = sid * _NC + cid

        def zrow(j, carry):
            for k in range(d_feat // _G):
                rowsf_v[0, j, pl.ds(k * _G, _G)] = jnp.zeros(
                    (_G,), jnp.float32)
            return carry

        lax.fori_loop(0, _C, zrow, 0, unroll=4)

        def zcopy(q, carry):
            pltpu.sync_copy(rowsf_v.at[0],
                            z_sh.at[pl.ds(sid * zrows + q * _C, _C)])
            return carry

        lax.fori_loop(0, nzc, zcopy, 0)
        if zrem:
            pltpu.sync_copy(
                rowsf_v.at[0, pl.ds(0, zrem)],
                z_sh.at[pl.ds(sid * zrows + nzc * _C, zrem)])
        plsc.subcore_barrier()

        def idx_start(c, bi):
            pltpu.async_copy(eidx_hbm.at[wid, c], idx_v.at[bi],
                             sem_i.at[bi])

        def idx_wait(c, bi):
            pltpu.make_async_copy(eidx_hbm.at[wid, c], idx_v.at[bi],
                                  sem_i.at[bi]).wait()

        def gather_start(c, b, bi):
            pltpu.async_copy(hp_hbm.at[idx_v.at[bi, 0]], rows_v.at[b],
                             sem_g.at[b])

        def gather_wait(c, b, bi):
            pltpu.make_async_copy(hp_hbm.at[idx_v.at[bi, 0]], rows_v.at[b],
                                  sem_g.at[b]).wait()

        def scatter_start(c, bf, bi):
            pltpu.async_copy(rowsf_v.at[bf], z_sh.at[idx_v.at[bi, 1]],
                             sem_s.at[bf], add=True)

        def scatter_wait(c, bf, bi):
            pltpu.make_async_copy(rowsf_v.at[bf], z_sh.at[idx_v.at[bi, 1]],
                                  sem_s.at[bf]).wait()

        dh = d_feat // 2

        def compute_chunk(c, b, bf, bi):
            def scale_row(j, c2):
                bev_i = plsc.load_gather(
                    idx_v, [jnp.full((_G,), bi, jnp.int32),
                            jnp.full((_G,), 2, jnp.int32),
                            jnp.full((_G,), j, jnp.int32)])
                bev = plsc.bitcast(bev_i, jnp.float32)
                for k in range(dh // _G):
                    w = rows_v[b, j, pl.ds(k * _G, _G)]
                    lo = plsc.bitcast(w << 16, jnp.float32)
                    hi = plsc.bitcast(w & jnp.int32(-65536), jnp.float32)
                    rowsf_v[bf, j, pl.ds(k * _G, _G)] = lo * bev
                    rowsf_v[bf, j, pl.ds(dh + k * _G, _G)] = hi * bev
                return c2

            lax.fori_loop(0, _C, scale_row, 0, unroll=4)

        def _pro_idx(c, carry):
            idx_start(c, c)
            return carry

        lax.fori_loop(0, min(_IPD, nchunk), _pro_idx, 0)

        def _pro_gather(c, carry):
            idx_wait(c, c)
            gather_start(c, c, c)
            return carry

        lax.fori_loop(0, min(_GPD, nchunk), _pro_gather, 0)

        def step(c, carry):
            b = lax.rem(c, _NBUF)
            bf = lax.rem(c, _NSC)
            bi = lax.rem(c, _NIB)

            @pl.when(c >= 2)
            def _drain():
                scatter_wait(c - 2, lax.rem(c - 2, _NSC),
                             lax.rem(c - 2, _NIB))

            @pl.when(c + _IPD < nchunk)
            def _refill():
                idx_start(c + _IPD, lax.rem(c + _IPD, _NIB))

            @pl.when(c + _GPD < nchunk)
            def _prefetch():
                bg = lax.rem(c + _GPD, _NBUF)
                big = lax.rem(c + _GPD, _NIB)
                idx_wait(c + _GPD, big)
                gather_start(c + _GPD, bg, big)

            gather_wait(c, b, bi)
            compute_chunk(c, b, bf, bi)
            scatter_start(c, bf, bi)
            return carry

        lax.fori_loop(0, nchunk, step, 0)
        def _final_drain(c, carry):
            scatter_wait(c, lax.rem(c, _NSC), lax.rem(c, _NIB))
            return carry

        lax.fori_loop(max(0, nchunk - 2), nchunk, _final_drain, 0)

        plsc.subcore_barrier()
        woff = pl.multiple_of(sid * rpt, 8)
        pltpu.sync_copy(z_sh.at[pl.ds(woff, rpt)],
                        out_hbm.at[cid, pl.ds(woff, rpt)])
        if rem:
            @pl.when(sid == _NS - 1)
            def _write_tail():
                pltpu.sync_copy(z_sh.at[pl.ds(rpt * _NS, rem)],
                                out_hbm.at[cid, pl.ds(rpt * _NS, rem)])

    return _scatter_kernel


def kernel(h, d, edge_index, gate_w, gate_b):
    n, d_feat = h.shape
    e = edge_index.shape[1]
    ept = e // _NW
    nchunk = ept // _C

    ei = edge_index.astype(jnp.int32).reshape(2 * e)

    w2r = gate_w.reshape(2, d_feat)
    bias2 = jnp.concatenate([gate_b, jnp.zeros((1,), jnp.float32)])
    bias2 = bias2.reshape(2, 1)

    abd, hp = pl.pallas_call(
        _ab_kernel,
        out_shape=[jax.ShapeDtypeStruct((3, n), jnp.float32),
                   jax.ShapeDtypeStruct((n, d_feat // 2), jnp.int32)],
    )(h, w2r, bias2, d.astype(jnp.float32))
    abd = abd.reshape(3 * n)

    mesh = plsc.VectorSubcoreMesh(core_axis_name="c", subcore_axis_name="s")

    eidx3 = pl.kernel(
        _make_gate_kernel(n, ept, e),
        out_type=jax.ShapeDtypeStruct((_NW, nchunk, 4, _C), jnp.int32),
        mesh=mesh,
        scratch_types=[
            pltpu.VMEM((3 * n,), jnp.float32),
            pltpu.VMEM((2 * ept,), jnp.int32),
            pltpu.VMEM((nchunk, 4, _C), jnp.int32),
            pltpu.SemaphoreType.DMA,
        ],
        compiler_params=pltpu.CompilerParams(needs_layout_passes=False),
    )(abd, ei)

    partials = pl.kernel(
        _make_scatter_kernel(n, d_feat, ept),
        out_type=jax.ShapeDtypeStruct((_NC, n, d_feat), jnp.float32),
        mesh=mesh,
        scratch_types=[
            pltpu.VMEM((_NIB, 4, _C), jnp.int32),
            pltpu.VMEM((_NBUF, _C, d_feat // 2), jnp.int32),
            pltpu.VMEM((_NSC, _C, d_feat), jnp.float32),
            pltpu.VMEM_SHARED((n, d_feat), jnp.float32),
            pltpu.SemaphoreType.DMA((_NIB,)),
            pltpu.SemaphoreType.DMA((_NBUF,)),
            pltpu.SemaphoreType.DMA((_NSC,)),
        ],
        compiler_params=pltpu.CompilerParams(
            needs_layout_passes=False, use_tc_tiling_on_sc=False),
    )(hp, eidx3)

    z = pl.pallas_call(
        _combine_kernel,
        out_shape=jax.ShapeDtypeStruct((n, d_feat), jnp.float32),
    )(partials)
    return z


# speedup vs baseline: 1.8983x; 1.8983x over previous
import jax
import jax.numpy as jnp
from jax import lax
from jax.experimental import pallas as pl
from jax.experimental.pallas import tpu as pltpu
from jax.experimental.pallas import tpu_sc as plsc

_NC = 2
_NS = 16
_NW = _NC * _NS
_C = 80
_G = 16
_NBUF = 4
_NIB = 8
_GPD = 2
_IPD = 6


def _ab_kernel(h_ref, w_ref, b_ref, d_ref, out_ref):
    ab = lax.dot_general(
        w_ref[...], h_ref[...], (((1,), (1,)), ((), ())),
        preferred_element_type=jnp.float32)
    out_ref[0:2, :] = ab + b_ref[...]
    out_ref[2, :] = d_ref[...]


def _combine_kernel(p_ref, out_ref):
    out_ref[...] = p_ref[0] + p_ref[1]


def _make_gate_kernel(n, ept, nedges):
    nchunk = ept // _C

    def _gate_kernel(abd_hbm, ei_hbm, out3_hbm, abd_v, idx_v, out_v, sem):
        cid = lax.axis_index("c")
        sid = lax.axis_index("s")
        wid = sid * _NC + cid

        off = pl.multiple_of(wid * ept, 8)
        cp1 = pltpu.make_async_copy(abd_hbm, abd_v, sem)
        cp2 = pltpu.make_async_copy(
            ei_hbm.at[pl.ds(off, ept)], idx_v.at[pl.ds(0, ept)], sem)
        cp3 = pltpu.make_async_copy(
            ei_hbm.at[pl.ds(nedges + off, ept)], idx_v.at[pl.ds(ept, ept)],
            sem)
        cp1.start()
        cp2.start()
        cp3.start()
        cp1.wait()
        cp2.wait()
        cp3.wait()

        def chunk(c, carry):
            for g in range(_C // _G):
                sl = pl.ds(g * _G, _G)
                si = idx_v[pl.ds(c * _C + g * _G, _G)]
                di = idx_v[pl.ds(ept + c * _C + g * _G, _G)]
                av = plsc.load_gather(abd_v, [di])
                bv = plsc.load_gather(abd_v, [si + n])
                dd = plsc.load_gather(abd_v, [di + 2 * n])
                ds_ = plsc.load_gather(abd_v, [si + 2 * n])
                u = jnp.exp(2.0 * (av + bv))
                t = 1.0 - 2.0 / (u + 1.0)
                ev = t * dd * ds_
                out_v[c, 0, sl] = si
                out_v[c, 1, sl] = di
                out_v[c, 2, sl] = plsc.bitcast(ev, jnp.int32)
            return carry

        lax.fori_loop(0, nchunk, chunk, 0, unroll=2)
        pltpu.sync_copy(out_v, out3_hbm.at[wid])

    return _gate_kernel


def _make_scatter_kernel(n, d_feat, ept):
    nchunk = ept // _C
    rpt = (n // _NS) // 8 * 8
    rem = n - rpt * _NS

    zrows = n // _NS
    nzc = zrows // _C
    zrem = zrows - nzc * _C

    def _scatter_kernel(h_hbm, eidx_hbm, out_hbm,
                        idx_v, rows_v, z_sh, sem_i, sem_g, sem_s):
        cid = lax.axis_index("c")
        sid = lax.axis_index("s")
        wid = sid * _NC + cid

        def zrow(j, carry):
            for k in range(d_feat // _G):
                rows_v[0, j, pl.ds(k * _G, _G)] = jnp.zeros(
                    (_G,), jnp.float32)
            return carry

        lax.fori_loop(0, _C, zrow, 0, unroll=4)

        def zcopy(q, carry):
            pltpu.sync_copy(rows_v.at[0],
                            z_sh.at[pl.ds(sid * zrows + q * _C, _C)])
            return carry

        lax.fori_loop(0, nzc, zcopy, 0)
        if zrem:
            pltpu.sync_copy(
                rows_v.at[0, pl.ds(0, zrem)],
                z_sh.at[pl.ds(sid * zrows + nzc * _C, zrem)])
        plsc.subcore_barrier()

        def idx_start(c, bi):
            pltpu.async_copy(eidx_hbm.at[wid, c], idx_v.at[bi],
                             sem_i.at[bi])

        def idx_wait(c, bi):
            pltpu.make_async_copy(eidx_hbm.at[wid, c], idx_v.at[bi],
                                  sem_i.at[bi]).wait()

        def gather_start(c, b, bi):
            pltpu.async_copy(h_hbm.at[idx_v.at[bi, 0]], rows_v.at[b],
                             sem_g.at[b])

        def gather_wait(c, b, bi):
            pltpu.make_async_copy(h_hbm.at[idx_v.at[bi, 0]], rows_v.at[b],
                                  sem_g.at[b]).wait()

        def scatter_start(c, b, bi):
            pltpu.async_copy(rows_v.at[b], z_sh.at[idx_v.at[bi, 1]],
                             sem_s.at[b], add=True)

        def scatter_wait(c, b, bi):
            pltpu.make_async_copy(rows_v.at[b], z_sh.at[idx_v.at[bi, 1]],
                                  sem_s.at[b]).wait()

        def compute_chunk(c, b, bi):
            def scale_row(j, c2):
                bev_i = plsc.load_gather(
                    idx_v, [jnp.full((_G,), bi, jnp.int32),
                            jnp.full((_G,), 2, jnp.int32),
                            jnp.full((_G,), j, jnp.int32)])
                bev = plsc.bitcast(bev_i, jnp.float32)
                for k in range(d_feat // _G):
                    sl = pl.ds(k * _G, _G)
                    rows_v[b, j, sl] = rows_v[b, j, sl] * bev
                return c2

            lax.fori_loop(0, _C, scale_row, 0, unroll=4)

        def _pro_idx(c, carry):
            idx_start(c, c)
            return carry

        lax.fori_loop(0, min(_IPD, nchunk), _pro_idx, 0)

        def _pro_gather(c, carry):
            idx_wait(c, c)
            gather_start(c, c, c)
            return carry

        lax.fori_loop(0, min(_GPD, nchunk), _pro_gather, 0)

        def step(c, carry):
            b = lax.rem(c, _NBUF)
            bi = lax.rem(c, _NIB)

            @pl.when(c >= 2)
            def _drain():
                scatter_wait(c - 2, lax.rem(c - 2, _NBUF),
                             lax.rem(c - 2, _NIB))

            @pl.when(c + _IPD < nchunk)
            def _refill():
                idx_start(c + _IPD, lax.rem(c + _IPD, _NIB))

            @pl.when(c + _GPD < nchunk)
            def _prefetch():
                bg = lax.rem(c + _GPD, _NBUF)
                big = lax.rem(c + _GPD, _NIB)
                idx_wait(c + _GPD, big)
                gather_start(c + _GPD, bg, big)

            gather_wait(c, b, bi)
            compute_chunk(c, b, bi)
            scatter_start(c, b, bi)
            return carry

        lax.fori_loop(0, nchunk, step, 0)
        def _final_drain(c, carry):
            scatter_wait(c, lax.rem(c, _NBUF), lax.rem(c, _NIB))
            return carry

        lax.fori_loop(max(0, nchunk - 2), nchunk, _final_drain, 0)

        plsc.subcore_barrier()
        woff = pl.multiple_of(sid * rpt, 8)
        pltpu.sync_copy(z_sh.at[pl.ds(woff, rpt)],
                        out_hbm.at[cid, pl.ds(woff, rpt)])
        if rem:
            @pl.when(sid == _NS - 1)
            def _write_tail():
                pltpu.sync_copy(z_sh.at[pl.ds(rpt * _NS, rem)],
                                out_hbm.at[cid, pl.ds(rpt * _NS, rem)])

    return _scatter_kernel


def kernel(h, d, edge_index, gate_w, gate_b):
    n, d_feat = h.shape
    e = edge_index.shape[1]
    ept = e // _NW
    nchunk = ept // _C

    ei = edge_index.astype(jnp.int32).reshape(2 * e)

    w2r = gate_w.reshape(2, d_feat)
    bias2 = jnp.concatenate([gate_b, jnp.zeros((1,), jnp.float32)])
    bias2 = bias2.reshape(2, 1)

    abd = pl.pallas_call(
        _ab_kernel,
        out_shape=jax.ShapeDtypeStruct((3, n), jnp.float32),
    )(h, w2r, bias2, d.astype(jnp.float32)).reshape(3 * n)

    mesh = plsc.VectorSubcoreMesh(core_axis_name="c", subcore_axis_name="s")

    eidx3 = pl.kernel(
        _make_gate_kernel(n, ept, e),
        out_type=jax.ShapeDtypeStruct((_NW, nchunk, 4, _C), jnp.int32),
        mesh=mesh,
        scratch_types=[
            pltpu.VMEM((3 * n,), jnp.float32),
            pltpu.VMEM((2 * ept,), jnp.int32),
            pltpu.VMEM((nchunk, 4, _C), jnp.int32),
            pltpu.SemaphoreType.DMA,
        ],
        compiler_params=pltpu.CompilerParams(needs_layout_passes=False),
    )(abd, ei)

    partials = pl.kernel(
        _make_scatter_kernel(n, d_feat, ept),
        out_type=jax.ShapeDtypeStruct((_NC, n, d_feat), jnp.float32),
        mesh=mesh,
        scratch_types=[
            pltpu.VMEM((_NIB, 4, _C), jnp.int32),
            pltpu.VMEM((_NBUF, _C, d_feat), jnp.float32),
            pltpu.VMEM_SHARED((n, d_feat), jnp.float32),
            pltpu.SemaphoreType.DMA((_NIB,)),
            pltpu.SemaphoreType.DMA((_NBUF,)),
            pltpu.SemaphoreType.DMA((_NBUF,)),
        ],
        compiler_params=pltpu.CompilerParams(needs_layout_passes=False),
    )(h, eidx3)

    z = pl.pallas_call(
        _combine_kernel,
        out_shape=jax.ShapeDtypeStruct((n, d_feat), jnp.float32),
    )(partials)
    return z
